# baseline (device time: 32836 ns/iter reference)
import jax
import jax.numpy as jnp
from jax import lax
from jax.experimental import pallas as pl
from jax.experimental.pallas import tpu as pltpu

N_Z = 4


def kernel(dy, W):
    m, k = dy.shape
    n = W.shape[0]

    def body(dy_ref, w_ref, out_ref, comm_ref, send_sems, recv_sems):
        my_x = lax.axis_index("x")
        my_y = lax.axis_index("y")
        my_z = lax.axis_index("z")
        left = (my_z - 1) % N_Z
        right = (my_z + 1) % N_Z

        barrier_sem = pltpu.get_barrier_semaphore()
        for nbr in (left, right):
            pl.semaphore_signal(
                barrier_sem,
                inc=1,
                device_id=(my_x, my_y, nbr),
                device_id_type=pl.DeviceIdType.MESH,
            )
        pl.semaphore_wait(barrier_sem, 2)

        dy_bf = dy_ref[...].astype(jnp.bfloat16)
        w_bf = w_ref[...].astype(jnp.bfloat16)
        partial = lax.dot_general(
            dy_bf, w_bf,
            (((1,), (1,)), ((), ())),
            preferred_element_type=jnp.float32,
        )
        out_ref[...] = partial
        comm_ref[0, :, :] = partial.astype(jnp.bfloat16)

        for h in range(N_Z - 1):
            send_slot = h % 2
            recv_slot = (h + 1) % 2
            rdma = pltpu.make_async_remote_copy(
                src_ref=comm_ref.at[send_slot],
                dst_ref=comm_ref.at[recv_slot],
                send_sem=send_sems.at[send_slot],
                recv_sem=recv_sems.at[recv_slot],
                device_id=(my_x, my_y, right),
                device_id_type=pl.DeviceIdType.MESH,
            )
            rdma.start()
            rdma.wait()
            out_ref[...] += comm_ref[recv_slot, :, :].astype(jnp.float32)

    return pl.pallas_call(
        body,
        out_shape=jax.ShapeDtypeStruct((m, n), jnp.float32),
        in_specs=[
            pl.BlockSpec(memory_space=pltpu.VMEM),
            pl.BlockSpec(memory_space=pltpu.VMEM),
        ],
        out_specs=pl.BlockSpec(memory_space=pltpu.VMEM),
        scratch_shapes=[
            pltpu.VMEM((2, m, n), jnp.bfloat16),
            pltpu.SemaphoreType.DMA((2,)),
            pltpu.SemaphoreType.DMA((2,)),
        ],
        compiler_params=pltpu.CompilerParams(collective_id=0),
    )(dy, W)


# device time: 23502 ns/iter; 1.3972x vs baseline; 1.3972x over previous
import jax
import jax.numpy as jnp
from jax import lax
from jax.experimental import pallas as pl
from jax.experimental.pallas import tpu as pltpu

N_Z = 4
BLK = 128
PIECE = 32


def kernel(dy, W):
    m, k = dy.shape
    n = W.shape[0]

    def body(dy_ref, w_ref, out_ref, part_ref, rs_ref, ag_ref,
             p1_send, p1_recv, p2_send, p2_recv, p3_send, p3_recv):
        my_x = lax.axis_index("x")
        my_y = lax.axis_index("y")
        my_z = lax.axis_index("z")
        my_q = 2 * my_x + my_y

        barrier_sem = pltpu.get_barrier_semaphore()
        for d in range(1, N_Z):
            pl.semaphore_signal(
                barrier_sem, inc=1,
                device_id=(my_x, my_y, (my_z + d) % N_Z),
                device_id_type=pl.DeviceIdType.MESH,
            )
        for a, b in ((0, 1), (1, 0), (1, 1)):
            pl.semaphore_signal(
                barrier_sem, inc=1,
                device_id=((my_x + a) % 2, (my_y + b) % 2, my_z),
                device_id_type=pl.DeviceIdType.MESH,
            )
        pl.semaphore_wait(barrier_sem, 6)

        dy_bf = dy_ref[pl.ds(my_q * BLK, BLK), :].astype(jnp.bfloat16)
        w_bf = w_ref[...].astype(jnp.bfloat16)
        partial = lax.dot_general(
            dy_bf, w_bf,
            (((1,), (1,)), ((), ())),
            preferred_element_type=jnp.float32,
        )
        part_ref[...] = partial.astype(jnp.bfloat16)

        p1 = []
        for d in range(1, N_Z):
            rdma = pltpu.make_async_remote_copy(
                src_ref=part_ref.at[pl.ds(((my_z + d) % N_Z) * PIECE, PIECE), :],
                dst_ref=rs_ref.at[d - 1],
                send_sem=p1_send.at[d - 1],
                recv_sem=p1_recv.at[d - 1],
                device_id=(my_x, my_y, (my_z + d) % N_Z),
                device_id_type=pl.DeviceIdType.MESH,
            )
            rdma.start()
            p1.append(rdma)
        for rdma in p1:
            rdma.wait()

        red = part_ref[pl.ds(my_z * PIECE, PIECE), :].astype(jnp.float32)
        for i in range(N_Z - 1):
            red += rs_ref[i].astype(jnp.float32)
        my_rows = my_q * BLK + my_z * PIECE
        ag_ref[pl.ds(my_rows, PIECE), :] = red.astype(jnp.bfloat16)

        p2 = []
        for d in range(1, N_Z):
            rdma = pltpu.make_async_remote_copy(
                src_ref=ag_ref.at[pl.ds(my_rows, PIECE), :],
                dst_ref=ag_ref.at[pl.ds(my_rows, PIECE), :],
                send_sem=p2_send.at[d - 1],
                recv_sem=p2_recv.at[d - 1],
                device_id=(my_x, my_y, (my_z + d) % N_Z),
                device_id_type=pl.DeviceIdType.MESH,
            )
            rdma.start()
            p2.append(rdma)
        for rdma in p2:
            rdma.wait()

        p3 = []
        for c, (a, b) in enumerate(((0, 1), (1, 0), (1, 1))):
            rdma = pltpu.make_async_remote_copy(
                src_ref=ag_ref.at[pl.ds(my_q * BLK, BLK), :],
                dst_ref=ag_ref.at[pl.ds(my_q * BLK, BLK), :],
                send_sem=p3_send.at[c],
                recv_sem=p3_recv.at[c],
                device_id=((my_x + a) % 2, (my_y + b) % 2, my_z),
                device_id_type=pl.DeviceIdType.MESH,
            )
            rdma.start()
            p3.append(rdma)
        for rdma in p3:
            rdma.wait()

        out_ref[...] = ag_ref[...].astype(jnp.float32)

    return pl.pallas_call(
        body,
        out_shape=jax.ShapeDtypeStruct((m, n), jnp.float32),
        in_specs=[
            pl.BlockSpec(memory_space=pltpu.VMEM),
            pl.BlockSpec(memory_space=pltpu.VMEM),
        ],
        out_specs=pl.BlockSpec(memory_space=pltpu.VMEM),
        scratch_shapes=[
            pltpu.VMEM((BLK, n), jnp.bfloat16),
            pltpu.VMEM((N_Z - 1, PIECE, n), jnp.bfloat16),
            pltpu.VMEM((m, n), jnp.bfloat16),
            pltpu.SemaphoreType.DMA((N_Z - 1,)),
            pltpu.SemaphoreType.DMA((N_Z - 1,)),
            pltpu.SemaphoreType.DMA((N_Z - 1,)),
            pltpu.SemaphoreType.DMA((N_Z - 1,)),
            pltpu.SemaphoreType.DMA((3,)),
            pltpu.SemaphoreType.DMA((3,)),
        ],
        compiler_params=pltpu.CompilerParams(collective_id=0),
    )(dy, W)
